# baseline (speedup 1.0000x reference)
import functools
import math

import jax
import jax.numpy as jnp
from jax import lax
from jax.experimental import pallas as pl
from jax.experimental.pallas import tpu as pltpu
from jax.experimental.pallas import tpu_sc as plsc

N = 2048
E = 16384
D = 64
DP = 128
NC = 2
NS = 16
NW = NC * NS
EPB = E // NW
NCHUNK = 4
CB = EPB // NCHUNK
ROWS_PER_TILE = N // NS


@functools.cache
def _sc_kernels():
    mesh = plsc.VectorSubcoreMesh(core_axis_name="c", subcore_axis_name="s")
    params = pltpu.CompilerParams(use_tc_tiling_on_sc=True)

    @functools.partial(
        pl.kernel,
        out_type=jax.ShapeDtypeStruct((E, DP), jnp.float32),
        mesh=mesh,
        compiler_params=params,
        scratch_types=[
            pltpu.VMEM((NCHUNK, CB), jnp.int32),
            pltpu.VMEM((CB, DP), jnp.float32),
            pltpu.SemaphoreType.DMA,
        ],
    )
    def gather(table_hbm, idx_hbm, out_hbm, idx_v, rows_v, sem):
        wid = lax.axis_index("s") * NC + lax.axis_index("c")
        base = wid * EPB
        pltpu.sync_copy(idx_hbm.at[wid], idx_v)
        for j in range(NCHUNK):
            pltpu.async_copy(table_hbm.at[idx_v.at[j]], rows_v, sem).wait()
            pltpu.sync_copy(rows_v, out_hbm.at[pl.ds(base + j * CB, CB)])

    @functools.partial(
        pl.kernel,
        out_type=[
            jax.ShapeDtypeStruct((NC, N, DP), jnp.float32),
            jax.ShapeDtypeStruct((NC, N, DP), jnp.float32),
        ],
        mesh=mesh,
        compiler_params=params,
        scratch_types=[
            pltpu.VMEM((NCHUNK, CB), jnp.int32),
            pltpu.VMEM((CB, DP), jnp.float32),
            pltpu.VMEM((CB, DP), jnp.float32),
            pltpu.VMEM_SHARED((N, DP), jnp.float32),
            pltpu.VMEM_SHARED((N, DP), jnp.float32),
        ],
    )
    def scatter_cnt(vals_hbm, idx_hbm, zeros_hbm, ones_hbm, out_hbm, cnt_hbm,
                    idx_v, vals_v, ones_v, acc_sh, cnt_sh):
        c = lax.axis_index("c")
        s = lax.axis_index("s")
        wid = s * NC + c
        base = wid * EPB
        rslice = pl.ds(s * ROWS_PER_TILE, ROWS_PER_TILE)
        pltpu.sync_copy(zeros_hbm.at[rslice], acc_sh.at[rslice])
        pltpu.sync_copy(zeros_hbm.at[rslice], cnt_sh.at[rslice])
        pltpu.sync_copy(idx_hbm.at[wid], idx_v)
        pltpu.sync_copy(ones_hbm, ones_v)
        plsc.subcore_barrier()
        for j in range(NCHUNK):
            pltpu.sync_copy(vals_hbm.at[pl.ds(base + j * CB, CB)], vals_v)
            pltpu.sync_copy(vals_v, acc_sh.at[idx_v.at[j]], add=True)
            pltpu.sync_copy(ones_v, cnt_sh.at[idx_v.at[j]], add=True)
        plsc.subcore_barrier()
        pltpu.sync_copy(acc_sh.at[rslice], out_hbm.at[c, rslice])
        pltpu.sync_copy(cnt_sh.at[rslice], cnt_hbm.at[c, rslice])

    @functools.partial(
        pl.kernel,
        out_type=jax.ShapeDtypeStruct((NC, N, DP), jnp.float32),
        mesh=mesh,
        compiler_params=params,
        scratch_types=[
            pltpu.VMEM((NCHUNK, CB), jnp.int32),
            pltpu.VMEM((CB, DP), jnp.float32),
            pltpu.VMEM_SHARED((N, DP), jnp.float32),
        ],
    )
    def scatter(vals_hbm, idx_hbm, zeros_hbm, out_hbm, idx_v, vals_v, acc_sh):
        c = lax.axis_index("c")
        s = lax.axis_index("s")
        wid = s * NC + c
        base = wid * EPB
        rslice = pl.ds(s * ROWS_PER_TILE, ROWS_PER_TILE)
        pltpu.sync_copy(zeros_hbm.at[rslice], acc_sh.at[rslice])
        pltpu.sync_copy(idx_hbm.at[wid], idx_v)
        plsc.subcore_barrier()
        for j in range(NCHUNK):
            pltpu.sync_copy(vals_hbm.at[pl.ds(base + j * CB, CB)], vals_v)
            pltpu.sync_copy(vals_v, acc_sh.at[idx_v.at[j]], add=True)
        plsc.subcore_barrier()
        pltpu.sync_copy(acc_sh.at[rslice], out_hbm.at[c, rslice])

    return gather, scatter_cnt, scatter


BE = 512
EG = E // BE
BE2 = 256
BR = 512
RG = N // BR


def _msg1_body(ax_ref, xj_ref, p_ref, o_ref):
    ax = ax_ref[...]
    xjT = jnp.transpose(xj_ref[...])
    acc = jnp.zeros((D, BE), jnp.float32)
    for i in range(D):
        z = jnp.dot(p_ref[i], ax, preferred_element_type=jnp.float32)
        acc = acc + jnp.broadcast_to(xjT[i:i + 1, :], (D, BE)) \
            * jnp.maximum(z, 0.0)
    o_ref[...] = jnp.concatenate(
        [jnp.transpose(acc), jnp.zeros((BE, DP - D), jnp.float32)], axis=1)


def _msg1_call(ax, xj, p):
    return pl.pallas_call(
        _msg1_body,
        grid=(EG,),
        in_specs=[
            pl.BlockSpec((2, BE), lambda i: (0, i)),
            pl.BlockSpec((BE, DP), lambda i: (i, 0)),
            pl.BlockSpec((D, D, 2), lambda i: (0, 0, 0)),
        ],
        out_specs=pl.BlockSpec((BE, DP), lambda i: (i, 0)),
        out_shape=jax.ShapeDtypeStruct((E, DP), jnp.float32),
    )(ax, xj, p)


def _msg2_body(ae_ref, xj_ref, p_ref, o_ref):
    z = jnp.dot(ae_ref[...], p_ref[...], preferred_element_type=jnp.float32)
    w2 = jnp.maximum(z, 0.0)
    d = jnp.sum(xj_ref[...] * w2, axis=1, keepdims=True)
    o_ref[...] = jnp.concatenate(
        [d, jnp.zeros((BE2, DP - 1), jnp.float32)], axis=1)


def _msg3_body(ae_ref, xj_ref, p_ref, o_ref):
    z = jnp.dot(ae_ref[...], p_ref[...], preferred_element_type=jnp.float32)
    o_ref[...] = xj_ref[...] * jnp.maximum(z, 0.0)


def _msg_call(body, ae, xj, p):
    specs = [
        pl.BlockSpec((BE2, 2), lambda i: (i, 0)),
        pl.BlockSpec((BE2, DP), lambda i: (i, 0)),
        pl.BlockSpec((2, DP), lambda i: (0, 0)),
    ]
    return pl.pallas_call(
        body,
        grid=(E // BE2,),
        in_specs=specs,
        out_specs=pl.BlockSpec((BE2, DP), lambda i: (i, 0)),
        out_shape=jax.ShapeDtypeStruct((E, DP), jnp.float32),
    )(ae, xj, p)


def _fin1_body(sp_ref, cp_ref, x_ref, root_ref, bias_ref, gs_ref, bb_ref,
               o_ref):
    s = (sp_ref[0] + sp_ref[1])[:, :D]
    cnt = cp_ref[0][:, 0:1] + cp_ref[1][:, 0:1]
    mean = s / jnp.maximum(cnt, 1.0)
    h = mean + jnp.dot(x_ref[...], root_ref[...],
                       preferred_element_type=jnp.float32) + bias_ref[...]
    z = h * gs_ref[...] + bb_ref[...]
    o_ref[...] = jnp.concatenate(
        [jax.nn.sigmoid(z), jnp.zeros((BR, DP - D), jnp.float32)], axis=1)


def _fin2_body(sp_ref, cp_ref, x1_ref, rootr_ref, bias_ref, gs_ref, bb_ref,
               o_ref):
    s2 = sp_ref[0][:, 0:1] + sp_ref[1][:, 0:1]
    cnt = cp_ref[0][:, 0:1] + cp_ref[1][:, 0:1]
    mean2 = s2 / jnp.maximum(cnt, 1.0)
    rr = jnp.sum(x1_ref[...] * rootr_ref[...], axis=1, keepdims=True)
    z = jnp.broadcast_to(mean2 + rr, (BR, DP))
    z = (z + bias_ref[...]) * gs_ref[...] + bb_ref[...]
    o_ref[...] = jax.nn.sigmoid(z)


def _fin3_body(sp_ref, cp_ref, x2b_ref, x1_ref, root_ref, bias_ref, gs_ref,
               bb_ref, o_ref):
    s = (sp_ref[0] + sp_ref[1])[:, :D]
    cnt = cp_ref[0][:, 0:1] + cp_ref[1][:, 0:1]
    mean = s / jnp.maximum(cnt, 1.0)
    xr = x2b_ref[...][:, 0:1] * root_ref[...]
    z = (mean + xr + bias_ref[...]) * gs_ref[...] + bb_ref[...]
    o_ref[...] = (jax.nn.sigmoid(z) + x1_ref[...][:, :D]) * 0.5


def _part_spec():
    return pl.BlockSpec((NC, BR, DP), lambda i: (0, i, 0))


def _vec_spec():
    return pl.BlockSpec((1, D), lambda i: (0, 0))


def _prow_spec():
    return pl.BlockSpec((BR, DP), lambda i: (i, 0))


def _fin_call(body, sp, cp, xa, xa_spec, mat, bias, gs, bb, mat_shape,
              out_w, vec_w=D):
    vs = pl.BlockSpec((1, vec_w), lambda i: (0, 0))
    return pl.pallas_call(
        body,
        grid=(RG,),
        in_specs=[
            _part_spec(), _part_spec(), xa_spec,
            pl.BlockSpec(mat_shape, lambda i: (0, 0)),
            vs, vs, vs,
        ],
        out_specs=pl.BlockSpec((BR, out_w), lambda i: (i, 0)),
        out_shape=jax.ShapeDtypeStruct((N, out_w), jnp.float32),
    )(sp, cp, xa, mat, bias, gs, bb)


def kernel(x, edge_index, edge_attr, W_nn1, b_nn1, root1, bias1, bn1_g, bn1_b,
           W_nn2, b_nn2, root2, bias2, bn2_g, bn2_b,
           W_nn3, b_nn3, root3, bias3, bn3_g, bn3_b):
    src3 = edge_index[0].reshape(NW, NCHUNK, CB)
    dst3 = edge_index[1].reshape(NW, NCHUNK, CB)
    a = edge_attr

    W1 = W_nn1.reshape(D, D)
    B1 = b_nn1.reshape(D, D)
    W2 = W_nn2.reshape(1, D)
    B2 = b_nn2.reshape(1, D)
    W3 = W_nn3.reshape(1, D)
    B3 = b_nn3.reshape(1, D)

    inv = 1.0 / math.sqrt(1.0 + 0.001)
    gs1 = (bn1_g * inv).reshape(1, D)
    bb1 = bn1_b.reshape(1, D)
    b1v = bias1.reshape(1, D)
    g2v = jnp.full((1, DP), bn2_g[0] * inv)
    bb2v = jnp.full((1, DP), bn2_b[0])
    b2v = jnp.full((1, DP), bias2[0])
    gs3 = (bn3_g * inv).reshape(1, D)
    bb3 = bn3_b.reshape(1, D)
    b3v = bias3.reshape(1, D)
    root2r = jnp.concatenate(
        [root2.reshape(1, D), jnp.zeros((1, DP - D), jnp.float32)], axis=1)
    root3r = root3.reshape(1, D)
    pad2 = jnp.zeros((2, DP - D), jnp.float32)
    p2 = jnp.concatenate([jnp.concatenate([W2, B2], axis=0), pad2], axis=1)
    p3 = jnp.concatenate([jnp.concatenate([W3, B3], axis=0), pad2], axis=1)
    aext = jnp.concatenate([a, jnp.ones((E, 1), jnp.float32)], axis=1)

    zeros_np = jnp.zeros((N, DP), jnp.float32)
    ones_cb = jnp.ones((CB, DP), jnp.float32)
    x_pad = jnp.concatenate([x, jnp.zeros((N, DP - D), jnp.float32)], axis=1)

    _gather, _scatter_cnt, _scatter = _sc_kernels()

    ax = jnp.concatenate([a.reshape(1, E), jnp.ones((1, E), jnp.float32)])
    p1 = jnp.stack([W1, B1], axis=-1)

    xj = _gather(x_pad, src3)
    m1 = _msg1_call(ax, xj, p1)
    s1p, cntp = _scatter_cnt(m1, dst3, zeros_np, ones_cb)
    x1 = _fin_call(_fin1_body, s1p, cntp, x,
                   pl.BlockSpec((BR, D), lambda i: (i, 0)),
                   root1, b1v, gs1, bb1, (D, D), DP)

    x1j = _gather(x1, src3)
    m2 = _msg_call(_msg2_body, aext, x1j, p2)
    s2p = _scatter(m2, dst3, zeros_np)
    x2b = _fin_call(_fin2_body, s2p, cntp, x1, _prow_spec(),
                    root2r, b2v, g2v, bb2v, (1, DP), DP, vec_w=DP)

    x2j = _gather(x2b, src3)
    m3 = _msg_call(_msg3_body, aext, x2j, p3)
    s3p = _scatter(m3, dst3, zeros_np)

    return pl.pallas_call(
        _fin3_body,
        grid=(RG,),
        in_specs=[
            _part_spec(), _part_spec(), _prow_spec(), _prow_spec(),
            _vec_spec(), _vec_spec(), _vec_spec(), _vec_spec(),
        ],
        out_specs=pl.BlockSpec((BR, D), lambda i: (i, 0)),
        out_shape=jax.ShapeDtypeStruct((N, D), jnp.float32),
    )(s3p, cntp, x2b, x1, root3r, b3v, gs3, bb3)


# speedup vs baseline: 1.3474x; 1.3474x over previous
import functools
import math

import jax
import jax.numpy as jnp
from jax import lax
from jax.experimental import pallas as pl
from jax.experimental.pallas import tpu as pltpu
from jax.experimental.pallas import tpu_sc as plsc

N = 2048
E = 16384
D = 64
DP = 128
NC = 2
NS = 16
NW = NC * NS
EPB = E // NW
NCHUNK = 4
CB = EPB // NCHUNK
ROWS_PER_TILE = N // NS


@functools.cache
def _sc_kernels():
    mesh = plsc.VectorSubcoreMesh(core_axis_name="c", subcore_axis_name="s")
    params = pltpu.CompilerParams(use_tc_tiling_on_sc=True)

    @functools.partial(
        pl.kernel,
        out_type=jax.ShapeDtypeStruct((E, DP), jnp.float32),
        mesh=mesh,
        compiler_params=params,
        scratch_types=[
            pltpu.VMEM((NCHUNK, CB), jnp.int32),
            pltpu.VMEM((CB, DP), jnp.float32),
            pltpu.SemaphoreType.DMA,
        ],
    )
    def gather(table_hbm, idx_hbm, out_hbm, idx_v, rows_v, sem):
        wid = lax.axis_index("s") * NC + lax.axis_index("c")
        base = wid * EPB
        pltpu.sync_copy(idx_hbm.at[wid], idx_v)
        for j in range(NCHUNK):
            pltpu.async_copy(table_hbm.at[idx_v.at[j]], rows_v, sem).wait()
            pltpu.sync_copy(rows_v, out_hbm.at[pl.ds(base + j * CB, CB)])

    @functools.partial(
        pl.kernel,
        out_type=[
            jax.ShapeDtypeStruct((NC, N, DP), jnp.float32),
            jax.ShapeDtypeStruct((NC, N, DP), jnp.float32),
        ],
        mesh=mesh,
        compiler_params=params,
        scratch_types=[
            pltpu.VMEM((NCHUNK, CB), jnp.int32),
            pltpu.VMEM((CB, DP), jnp.float32),
            pltpu.VMEM((CB, DP), jnp.float32),
            pltpu.VMEM_SHARED((N, DP), jnp.float32),
            pltpu.VMEM_SHARED((N, DP), jnp.float32),
        ],
    )
    def scatter_cnt(vals_hbm, idx_hbm, zeros_hbm, ones_hbm, out_hbm, cnt_hbm,
                    idx_v, vals_v, ones_v, acc_sh, cnt_sh):
        c = lax.axis_index("c")
        s = lax.axis_index("s")
        wid = s * NC + c
        base = wid * EPB
        rslice = pl.ds(s * ROWS_PER_TILE, ROWS_PER_TILE)
        pltpu.sync_copy(zeros_hbm.at[rslice], acc_sh.at[rslice])
        pltpu.sync_copy(zeros_hbm.at[rslice], cnt_sh.at[rslice])
        pltpu.sync_copy(idx_hbm.at[wid], idx_v)
        pltpu.sync_copy(ones_hbm, ones_v)
        plsc.subcore_barrier()
        for j in range(NCHUNK):
            pltpu.sync_copy(vals_hbm.at[pl.ds(base + j * CB, CB)], vals_v)
            pltpu.sync_copy(vals_v, acc_sh.at[idx_v.at[j]], add=True)
            pltpu.sync_copy(ones_v, cnt_sh.at[idx_v.at[j]], add=True)
        plsc.subcore_barrier()
        pltpu.sync_copy(acc_sh.at[rslice], out_hbm.at[c, rslice])
        pltpu.sync_copy(cnt_sh.at[rslice], cnt_hbm.at[c, rslice])

    @functools.partial(
        pl.kernel,
        out_type=jax.ShapeDtypeStruct((NC, N, DP), jnp.float32),
        mesh=mesh,
        compiler_params=params,
        scratch_types=[
            pltpu.VMEM((NCHUNK, CB), jnp.int32),
            pltpu.VMEM((CB, DP), jnp.float32),
            pltpu.VMEM_SHARED((N, DP), jnp.float32),
        ],
    )
    def scatter(vals_hbm, idx_hbm, zeros_hbm, out_hbm, idx_v, vals_v, acc_sh):
        c = lax.axis_index("c")
        s = lax.axis_index("s")
        wid = s * NC + c
        base = wid * EPB
        rslice = pl.ds(s * ROWS_PER_TILE, ROWS_PER_TILE)
        pltpu.sync_copy(zeros_hbm.at[rslice], acc_sh.at[rslice])
        pltpu.sync_copy(idx_hbm.at[wid], idx_v)
        plsc.subcore_barrier()
        for j in range(NCHUNK):
            pltpu.sync_copy(vals_hbm.at[pl.ds(base + j * CB, CB)], vals_v)
            pltpu.sync_copy(vals_v, acc_sh.at[idx_v.at[j]], add=True)
        plsc.subcore_barrier()
        pltpu.sync_copy(acc_sh.at[rslice], out_hbm.at[c, rslice])

    return gather, scatter_cnt, scatter


BE = 512
EG = E // BE
BE2 = 2048
BR = 512
RG = N // BR


def _msg1_body(ax_ref, xj_ref, p_ref, o_ref):
    ax = ax_ref[...]
    xjT = jnp.transpose(xj_ref[...])
    acc = jnp.zeros((D, BE), jnp.float32)
    for i in range(D):
        z = jnp.dot(p_ref[i], ax, preferred_element_type=jnp.float32)
        acc = acc + jnp.broadcast_to(xjT[i:i + 1, :], (D, BE)) \
            * jnp.maximum(z, 0.0)
    o_ref[...] = jnp.concatenate(
        [jnp.transpose(acc), jnp.zeros((BE, DP - D), jnp.float32)], axis=1)


def _msg1_call(ax, xj, p):
    return pl.pallas_call(
        _msg1_body,
        grid=(EG,),
        in_specs=[
            pl.BlockSpec((2, BE), lambda i: (0, i)),
            pl.BlockSpec((BE, DP), lambda i: (i, 0)),
            pl.BlockSpec((D, D, 2), lambda i: (0, 0, 0)),
        ],
        out_specs=pl.BlockSpec((BE, DP), lambda i: (i, 0)),
        out_shape=jax.ShapeDtypeStruct((E, DP), jnp.float32),
    )(ax, xj, p)


def _msg2_body(ax_ref, xj_ref, p_ref, o_ref):
    z = jnp.dot(jnp.transpose(ax_ref[...]), p_ref[...],
                preferred_element_type=jnp.float32)
    w2 = jnp.maximum(z, 0.0)
    d = jnp.sum(xj_ref[...] * w2, axis=1, keepdims=True)
    o_ref[...] = jnp.concatenate(
        [d, jnp.zeros((BE2, DP - 1), jnp.float32)], axis=1)


def _msg3_body(ax_ref, xj_ref, p_ref, o_ref):
    z = jnp.dot(jnp.transpose(ax_ref[...]), p_ref[...],
                preferred_element_type=jnp.float32)
    o_ref[...] = xj_ref[...] * jnp.maximum(z, 0.0)


def _msg_call(body, ae, xj, p):
    specs = [
        pl.BlockSpec((2, BE2), lambda i: (0, i)),
        pl.BlockSpec((BE2, DP), lambda i: (i, 0)),
        pl.BlockSpec((2, DP), lambda i: (0, 0)),
    ]
    return pl.pallas_call(
        body,
        grid=(E // BE2,),
        in_specs=specs,
        out_specs=pl.BlockSpec((BE2, DP), lambda i: (i, 0)),
        out_shape=jax.ShapeDtypeStruct((E, DP), jnp.float32),
    )(ae, xj, p)


def _fin1_body(sp_ref, cp_ref, x_ref, root_ref, bias_ref, gs_ref, bb_ref,
               o_ref):
    s = (sp_ref[0] + sp_ref[1])[:, :D]
    cnt = cp_ref[0][:, 0:1] + cp_ref[1][:, 0:1]
    mean = s / jnp.maximum(cnt, 1.0)
    h = mean + jnp.dot(x_ref[...], root_ref[...],
                       preferred_element_type=jnp.float32) + bias_ref[...]
    z = h * gs_ref[...] + bb_ref[...]
    o_ref[...] = jnp.concatenate(
        [jax.nn.sigmoid(z), jnp.zeros((BR, DP - D), jnp.float32)], axis=1)


def _fin2_body(sp_ref, cp_ref, x1_ref, rootr_ref, bias_ref, gs_ref, bb_ref,
               o_ref):
    s2 = sp_ref[0][:, 0:1] + sp_ref[1][:, 0:1]
    cnt = cp_ref[0][:, 0:1] + cp_ref[1][:, 0:1]
    mean2 = s2 / jnp.maximum(cnt, 1.0)
    rr = jnp.sum(x1_ref[...] * rootr_ref[...], axis=1, keepdims=True)
    z = jnp.broadcast_to(mean2 + rr, (BR, DP))
    z = (z + bias_ref[...]) * gs_ref[...] + bb_ref[...]
    o_ref[...] = jax.nn.sigmoid(z)


def _fin3_body(sp_ref, cp_ref, x2b_ref, x1_ref, root_ref, bias_ref, gs_ref,
               bb_ref, o_ref):
    s = (sp_ref[0] + sp_ref[1])[:, :D]
    cnt = cp_ref[0][:, 0:1] + cp_ref[1][:, 0:1]
    mean = s / jnp.maximum(cnt, 1.0)
    xr = x2b_ref[...][:, 0:1] * root_ref[...]
    z = (mean + xr + bias_ref[...]) * gs_ref[...] + bb_ref[...]
    o_ref[...] = (jax.nn.sigmoid(z) + x1_ref[...][:, :D]) * 0.5


def _part_spec():
    return pl.BlockSpec((NC, BR, DP), lambda i: (0, i, 0))


def _vec_spec():
    return pl.BlockSpec((1, D), lambda i: (0, 0))


def _prow_spec():
    return pl.BlockSpec((BR, DP), lambda i: (i, 0))


def _fin_call(body, sp, cp, xa, xa_spec, mat, bias, gs, bb, mat_shape,
              out_w, vec_w=D):
    vs = pl.BlockSpec((1, vec_w), lambda i: (0, 0))
    return pl.pallas_call(
        body,
        grid=(RG,),
        in_specs=[
            _part_spec(), _part_spec(), xa_spec,
            pl.BlockSpec(mat_shape, lambda i: (0, 0)),
            vs, vs, vs,
        ],
        out_specs=pl.BlockSpec((BR, out_w), lambda i: (i, 0)),
        out_shape=jax.ShapeDtypeStruct((N, out_w), jnp.float32),
    )(sp, cp, xa, mat, bias, gs, bb)


def kernel(x, edge_index, edge_attr, W_nn1, b_nn1, root1, bias1, bn1_g, bn1_b,
           W_nn2, b_nn2, root2, bias2, bn2_g, bn2_b,
           W_nn3, b_nn3, root3, bias3, bn3_g, bn3_b):
    src3 = edge_index[0].reshape(NW, NCHUNK, CB)
    dst3 = edge_index[1].reshape(NW, NCHUNK, CB)
    a = edge_attr

    W1 = W_nn1.reshape(D, D)
    B1 = b_nn1.reshape(D, D)
    W2 = W_nn2.reshape(1, D)
    B2 = b_nn2.reshape(1, D)
    W3 = W_nn3.reshape(1, D)
    B3 = b_nn3.reshape(1, D)

    inv = 1.0 / math.sqrt(1.0 + 0.001)
    gs1 = (bn1_g * inv).reshape(1, D)
    bb1 = bn1_b.reshape(1, D)
    b1v = bias1.reshape(1, D)
    g2v = jnp.full((1, DP), bn2_g[0] * inv)
    bb2v = jnp.full((1, DP), bn2_b[0])
    b2v = jnp.full((1, DP), bias2[0])
    gs3 = (bn3_g * inv).reshape(1, D)
    bb3 = bn3_b.reshape(1, D)
    b3v = bias3.reshape(1, D)
    root2r = jnp.concatenate(
        [root2.reshape(1, D), jnp.zeros((1, DP - D), jnp.float32)], axis=1)
    root3r = root3.reshape(1, D)
    pad2 = jnp.zeros((2, DP - D), jnp.float32)
    p2 = jnp.concatenate([jnp.concatenate([W2, B2], axis=0), pad2], axis=1)
    p3 = jnp.concatenate([jnp.concatenate([W3, B3], axis=0), pad2], axis=1)

    zeros_np = jnp.zeros((N, DP), jnp.float32)
    ones_cb = jnp.ones((CB, DP), jnp.float32)
    x_pad = jnp.concatenate([x, jnp.zeros((N, DP - D), jnp.float32)], axis=1)

    _gather, _scatter_cnt, _scatter = _sc_kernels()

    ax = jnp.concatenate([a.reshape(1, E), jnp.ones((1, E), jnp.float32)])
    p1 = jnp.stack([W1, B1], axis=-1)

    xj = _gather(x_pad, src3)
    m1 = _msg1_call(ax, xj, p1)
    s1p, cntp = _scatter_cnt(m1, dst3, zeros_np, ones_cb)
    x1 = _fin_call(_fin1_body, s1p, cntp, x,
                   pl.BlockSpec((BR, D), lambda i: (i, 0)),
                   root1, b1v, gs1, bb1, (D, D), DP)

    x1j = _gather(x1, src3)
    m2 = _msg_call(_msg2_body, ax, x1j, p2)
    s2p = _scatter(m2, dst3, zeros_np)
    x2b = _fin_call(_fin2_body, s2p, cntp, x1, _prow_spec(),
                    root2r, b2v, g2v, bb2v, (1, DP), DP, vec_w=DP)

    x2j = _gather(x2b, src3)
    m3 = _msg_call(_msg3_body, ax, x2j, p3)
    s3p = _scatter(m3, dst3, zeros_np)

    return pl.pallas_call(
        _fin3_body,
        grid=(RG,),
        in_specs=[
            _part_spec(), _part_spec(), _prow_spec(), _prow_spec(),
            _vec_spec(), _vec_spec(), _vec_spec(), _vec_spec(),
        ],
        out_specs=pl.BlockSpec((BR, D), lambda i: (i, 0)),
        out_shape=jax.ShapeDtypeStruct((N, D), jnp.float32),
    )(s3p, cntp, x2b, x1, root3r, b3v, gs3, bb3)
